# baseline (speedup 1.0000x reference)
import functools

import jax
import jax.numpy as jnp
from jax.experimental import pallas as pl
from jax.experimental.pallas import tpu as pltpu

_K = 100
_BM = 512


def _tree_count(m_bool):
    s = jnp.where(m_bool, jnp.bfloat16(1), jnp.bfloat16(0))
    while s.shape[1] > 128:
        h = s.shape[1] // 2
        s = s[:, :h] + s[:, h:]
    return jnp.sum(s.astype(jnp.float32), axis=1, keepdims=True)


def _split16(bits):
    y16 = jax.lax.bitcast_convert_type(
        jnp.bitwise_and(bits, jnp.int32(-65536)), jnp.float32
    ).astype(jnp.bfloat16)
    lo = (jnp.bitwise_xor(bits, 0x8000) & 0xFFFF).astype(jnp.int16)
    return y16, lo


def _body(x_ref, we_ref, be_ref, wd_ref, bd_ref, o_ref, y16_ref, lo_ref):
    s = pl.program_id(0)
    np_ = pl.num_programs(0)
    nb = np_ - 2
    wr = jax.lax.rem(s, 2)
    rd = 1 - wr

    def encode_chunk(j, xm):
        js = pl.multiple_of(j * 256, 256)
        pr = jnp.dot(
            xm, we_ref[:, pl.ds(js, 256)], preferred_element_type=jnp.float32
        )
        yc = jnp.maximum(pr + be_ref[:, pl.ds(js, 256)], 0.0)
        y16c, loc = _split16(jax.lax.bitcast_convert_type(yc, jnp.int32))
        y16_ref[wr, :, pl.ds(js, 256)] = y16c
        lo_ref[wr, :, pl.ds(js, 256)] = loc

    def decode_chunk(j, src):
        js = pl.multiple_of(j * 256, 256)
        o_ref[:, pl.ds(js, 256)] = (
            jnp.dot(
                src, wd_ref[:, pl.ds(js, 256)], preferred_element_type=jnp.float32
            )
            + bd_ref[:, pl.ds(js, 256)]
        )

    @pl.when(s == 0)
    def _prologue():
        xm = (x_ref[...] - bd_ref[...]).astype(jnp.bfloat16)
        for j in range(16):
            encode_chunk(j, xm)

    @pl.when(jnp.logical_and(s >= 1, s <= nb))
    def _compute():
        xm = (x_ref[...] - bd_ref[...]).astype(jnp.bfloat16)
        y16 = y16_ref[rd]
        lo = lo_ref[rd]
        kf = jnp.float32(_K)

        def step1(i, t):
            cand = jnp.bitwise_or(t, jax.lax.shift_left(1, 14 - i))
            cand_b = jax.lax.bitcast_convert_type(
                cand.astype(jnp.int16), jnp.bfloat16
            )
            cnt = _tree_count(y16 >= cand_b)
            return jnp.where(cnt >= kf, cand, t)

        t1 = jax.lax.fori_loop(0, 7, step1, jnp.zeros((_BM, 1), jnp.int32))
        enc_prev = y16_ref[wr]
        for i in range(7, 9):
            decode_chunk(2 * (i - 7), enc_prev)
            decode_chunk(2 * (i - 7) + 1, enc_prev)
            t1 = step1(i, t1)
        enc_sched = [3, 3, 3, 3, 2, 2]
        nxt = 0
        for k, i in enumerate(range(9, 15)):
            for _ in range(enc_sched[k]):
                encode_chunk(nxt, xm)
                nxt += 1
            t1 = step1(i, t1)

        t1_b = jax.lax.bitcast_convert_type(t1.astype(jnp.int16), jnp.bfloat16)
        n_gt = _tree_count(y16 > t1_b)
        meq = y16 == t1_b

        def step2(i, t):
            cand = jnp.bitwise_or(t, jax.lax.shift_left(1, 15 - i))
            cand16 = jnp.bitwise_xor(cand, 0x8000).astype(jnp.int16)
            cnt = n_gt + _tree_count((lo >= cand16) & meq)
            return jnp.where(cnt >= kf, cand, t)

        u = jax.lax.fori_loop(0, 16, step2, jnp.zeros((_BM, 1), jnp.int32))
        thr = jnp.bitwise_or(jax.lax.shift_left(t1, 16), u)

        for j in range(4):
            js = 1024 * j
            hi = jax.lax.bitcast_convert_type(
                y16[:, js : js + 1024], jnp.int16
            ).astype(jnp.int32)
            lor = jnp.bitwise_xor(
                jnp.bitwise_and(lo[:, js : js + 1024].astype(jnp.int32), 0xFFFF),
                0x8000,
            )
            bits = jnp.bitwise_or(jax.lax.shift_left(hi, 16), lor)
            yv = jax.lax.bitcast_convert_type(bits, jnp.float32)
            y16_ref[rd, :, js : js + 1024] = jnp.where(
                bits >= thr, yv, 0.0
            ).astype(jnp.bfloat16)

    @pl.when(s == np_ - 1)
    def _tail():
        for j in range(4):
            decode_chunk(j, y16_ref[wr])


@jax.jit
def kernel(x, W_enc, b_enc, W_dec, b_dec):
    B, d_in = x.shape
    d_sae = W_enc.shape[1]
    nb = B // _BM
    be = b_enc.reshape(1, d_sae)
    bd = b_dec.reshape(1, d_in)
    return pl.pallas_call(
        _body,
        grid=(nb + 2,),
        in_specs=[
            pl.BlockSpec((_BM, d_in), lambda i: (jnp.minimum(i, nb - 1), 0)),
            pl.BlockSpec((d_in, d_sae), lambda i: (0, 0)),
            pl.BlockSpec((1, d_sae), lambda i: (0, 0)),
            pl.BlockSpec((d_sae, d_in), lambda i: (0, 0)),
            pl.BlockSpec((1, d_in), lambda i: (0, 0)),
        ],
        out_specs=pl.BlockSpec(
            (_BM, d_in), lambda i: (jnp.maximum(i - 2, 0), 0)
        ),
        out_shape=jax.ShapeDtypeStruct((B, d_in), jnp.float32),
        scratch_shapes=[
            pltpu.VMEM((2, _BM, d_sae), jnp.bfloat16),
            pltpu.VMEM((2, _BM, d_sae), jnp.int16),
        ],
    )(x, W_enc.astype(jnp.bfloat16), be, W_dec.astype(jnp.bfloat16), bd)


# speedup vs baseline: 1.0505x; 1.0505x over previous
import functools

import jax
import jax.numpy as jnp
from jax.experimental import pallas as pl
from jax.experimental.pallas import tpu as pltpu

_K = 100
_BM = 512


def _tree_count(m_bool):
    s = jnp.where(m_bool, jnp.bfloat16(1), jnp.bfloat16(0))
    while s.shape[1] > 128:
        h = s.shape[1] // 2
        s = s[:, :h] + s[:, h:]
    return jnp.sum(s.astype(jnp.float32), axis=1, keepdims=True)


def _body(x_ref, we_ref, be_ref, wd_ref, bd_ref, o_ref, enc_ref):
    s = pl.program_id(0)
    nb = pl.num_programs(0) - 1
    cur = jax.lax.rem(s, 2)
    prv = 1 - cur

    def decode_chunk(j, src):
        js = pl.multiple_of(j * 256, 256)
        o_ref[:, pl.ds(js, 256)] = (
            jnp.dot(
                src, wd_ref[:, pl.ds(js, 256)], preferred_element_type=jnp.float32
            )
            + bd_ref[:, pl.ds(js, 256)]
        )

    @pl.when(s < nb)
    def _compute():
        xm = (x_ref[...] - bd_ref[...]).astype(jnp.bfloat16)
        pre = jnp.dot(xm, we_ref[...], preferred_element_type=jnp.float32)
        y = jnp.maximum(pre + be_ref[...], 0.0)
        bits = jax.lax.bitcast_convert_type(y, jnp.int32)
        bm = y.shape[0]
        kf = jnp.float32(_K)

        y16 = jax.lax.bitcast_convert_type(
            jnp.bitwise_and(bits, jnp.int32(-65536)), jnp.float32
        ).astype(jnp.bfloat16)
        lo = (jnp.bitwise_xor(bits, 0x8000) & 0xFFFF).astype(jnp.int16)

        def step1(i, t):
            cand = jnp.bitwise_or(t, jax.lax.shift_left(1, 14 - i))
            cand_b = jax.lax.bitcast_convert_type(
                cand.astype(jnp.int16), jnp.bfloat16
            )
            cnt = _tree_count(y16 >= cand_b)
            return jnp.where(cnt >= kf, cand, t)

        t1 = jax.lax.fori_loop(0, 15, step1, jnp.zeros((bm, 1), jnp.int32))
        t1_b = jax.lax.bitcast_convert_type(t1.astype(jnp.int16), jnp.bfloat16)
        n_gt = _tree_count(y16 > t1_b)
        meq = y16 == t1_b

        def step2(i, t):
            cand = jnp.bitwise_or(t, jax.lax.shift_left(1, 15 - i))
            cand16 = jnp.bitwise_xor(cand, 0x8000).astype(jnp.int16)
            cnt = n_gt + _tree_count((lo >= cand16) & meq)
            return jnp.where(cnt >= kf, cand, t)

        u = jax.lax.fori_loop(0, 12, step2, jnp.zeros((bm, 1), jnp.int32))
        enc_prev = enc_ref[prv]
        for i in range(12, 16):
            decode_chunk(i - 12, enc_prev)
            u = step2(i, u)

        thr = jnp.bitwise_or(jax.lax.shift_left(t1, 16), u)
        enc_ref[cur] = jnp.where(bits >= thr, y, 0.0).astype(jnp.bfloat16)

    @pl.when(s == nb)
    def _tail():
        for j in range(4):
            decode_chunk(j, enc_ref[prv])


@jax.jit
def kernel(x, W_enc, b_enc, W_dec, b_dec):
    B, d_in = x.shape
    d_sae = W_enc.shape[1]
    nb = B // _BM
    be = b_enc.reshape(1, d_sae)
    bd = b_dec.reshape(1, d_in)
    return pl.pallas_call(
        _body,
        grid=(nb + 1,),
        in_specs=[
            pl.BlockSpec((_BM, d_in), lambda i: (jnp.minimum(i, nb - 1), 0)),
            pl.BlockSpec((d_in, d_sae), lambda i: (0, 0)),
            pl.BlockSpec((1, d_sae), lambda i: (0, 0)),
            pl.BlockSpec((d_sae, d_in), lambda i: (0, 0)),
            pl.BlockSpec((1, d_in), lambda i: (0, 0)),
        ],
        out_specs=pl.BlockSpec(
            (_BM, d_in), lambda i: (jnp.maximum(i - 1, 0), 0)
        ),
        out_shape=jax.ShapeDtypeStruct((B, d_in), jnp.float32),
        scratch_shapes=[pltpu.VMEM((2, _BM, d_sae), jnp.bfloat16)],
    )(x, W_enc.astype(jnp.bfloat16), be, W_dec.astype(jnp.bfloat16), bd)
